# baseline (speedup 1.0000x reference)
import functools

import jax
import jax.numpy as jnp
import numpy as np
from jax import lax
from jax.experimental import pallas as pl
from jax.experimental.pallas import tpu as pltpu
from jax.experimental.pallas import tpu_sc as plsc

_ACCEL_FACTOR = 4
_NUM_CENTRAL_LINES = 30
_ZERO_FILL_WIDTH = 368


def _ky_positions():
    center = _ZERO_FILL_WIDTH // 2
    half_width = _NUM_CENTRAL_LINES // 2
    central = np.arange(center - half_width,
                        center + half_width + _NUM_CENTRAL_LINES % 2)
    accel = np.arange(_ZERO_FILL_WIDTH)[::_ACCEL_FACTOR]
    accel = accel[~np.isin(accel, central)]
    return np.sort(np.concatenate([central, accel]))


_KY = _ky_positions()
_NUM_KY = _KY.shape[0]

_ROWS = 32 * 15 * 320
_NW = 32
_RPW = _ROWS // _NW
_C = 96
_NCHUNK = _RPW // _C
_IN_W = _C * _NUM_KY
_OUT_W = _C * _ZERO_FILL_WIDTH


def _chunk_scatter_indices():
    r = np.arange(_IN_W) // _NUM_KY
    j = np.arange(_IN_W) % _NUM_KY
    return jnp.asarray(_ZERO_FILL_WIDTH * r + _KY[j], dtype=jnp.int32)


@jax.jit
def _sc_zero_fill(x_flat, idx_flat):
    mesh = plsc.VectorSubcoreMesh(core_axis_name="c", subcore_axis_name="s")

    @functools.partial(
        pl.kernel,
        mesh=mesh,
        out_type=jax.ShapeDtypeStruct((_ROWS * _ZERO_FILL_WIDTH,), jnp.float32),
        compiler_params=pltpu.CompilerParams(needs_layout_passes=False),
        scratch_types=[
            pltpu.VMEM((_IN_W,), jnp.float32),
            pltpu.VMEM((_OUT_W,), jnp.float32),
            pltpu.VMEM((_IN_W,), jnp.int32),
        ],
    )
    def k(x_hbm, idx_hbm, out_hbm, in_v, out_v, idx_v):
        wid = lax.axis_index("s") * 2 + lax.axis_index("c")
        base_in = wid * (_RPW * _NUM_KY)
        base_out = wid * (_RPW * _ZERO_FILL_WIDTH)

        pltpu.sync_copy(idx_hbm, idx_v)

        zeros = jnp.zeros((16,), jnp.float32)

        def zbody(i, carry):
            out_v[pl.ds(i * 16, 16)] = zeros
            return carry

        lax.fori_loop(0, _OUT_W // 16, zbody, 0)

        def chunk(c, carry):
            pltpu.sync_copy(x_hbm.at[pl.ds(base_in + c * _IN_W, _IN_W)], in_v)

            def body(i, inner):
                v = in_v[pl.ds(i * 16, 16)]
                ix = idx_v[pl.ds(i * 16, 16)]
                plsc.store_scatter(out_v, [ix], v)
                return inner

            lax.fori_loop(0, _IN_W // 16, body, 0)
            pltpu.sync_copy(out_v,
                            out_hbm.at[pl.ds(base_out + c * _OUT_W, _OUT_W)])
            return carry

        lax.fori_loop(0, _NCHUNK, chunk, 0)

    return k(x_flat, idx_flat)


def kernel(undersampled_ksp):
    lead = undersampled_ksp.shape[:-1]
    x_flat = undersampled_ksp.reshape(-1)
    out = _sc_zero_fill(x_flat, _chunk_scatter_indices())
    return out.reshape(*lead, _ZERO_FILL_WIDTH)


# speedup vs baseline: 1.4770x; 1.4770x over previous
import functools

import jax
import jax.numpy as jnp
import numpy as np
from jax import lax
from jax.experimental import pallas as pl
from jax.experimental.pallas import tpu as pltpu
from jax.experimental.pallas import tpu_sc as plsc

_ACCEL_FACTOR = 4
_NUM_CENTRAL_LINES = 30
_ZERO_FILL_WIDTH = 368


def _ky_positions():
    center = _ZERO_FILL_WIDTH // 2
    half_width = _NUM_CENTRAL_LINES // 2
    central = np.arange(center - half_width,
                        center + half_width + _NUM_CENTRAL_LINES % 2)
    accel = np.arange(_ZERO_FILL_WIDTH)[::_ACCEL_FACTOR]
    accel = accel[~np.isin(accel, central)]
    return np.sort(np.concatenate([central, accel]))


_KY = _ky_positions()
_NUM_KY = _KY.shape[0]

_ROWS = 32 * 15 * 320
_NW = 32
_RPW = _ROWS // _NW
_C = 96
_NCHUNK = _RPW // _C
_IN_W = _C * _NUM_KY
_OUT_W = _C * _ZERO_FILL_WIDTH


def _chunk_scatter_indices():
    r = np.arange(_IN_W) // _NUM_KY
    j = np.arange(_IN_W) % _NUM_KY
    return jnp.asarray(_ZERO_FILL_WIDTH * r + _KY[j], dtype=jnp.int32)


@jax.jit
def _sc_zero_fill(x_flat, idx_flat):
    mesh = plsc.VectorSubcoreMesh(core_axis_name="c", subcore_axis_name="s")

    @functools.partial(
        pl.kernel,
        mesh=mesh,
        out_type=jax.ShapeDtypeStruct((_ROWS * _ZERO_FILL_WIDTH,), jnp.float32),
        compiler_params=pltpu.CompilerParams(needs_layout_passes=False),
        scratch_types=[
            pltpu.VMEM((_IN_W,), jnp.float32),
            pltpu.VMEM((_IN_W,), jnp.float32),
            pltpu.VMEM((_OUT_W,), jnp.float32),
            pltpu.VMEM((_OUT_W,), jnp.float32),
            pltpu.VMEM((_IN_W,), jnp.int32),
            pltpu.SemaphoreType.DMA,
            pltpu.SemaphoreType.DMA,
            pltpu.SemaphoreType.DMA,
            pltpu.SemaphoreType.DMA,
        ],
    )
    def k(x_hbm, idx_hbm, out_hbm, in_v0, in_v1, out_v0, out_v1, idx_v,
          si0, si1, so0, so1):
        wid = lax.axis_index("s") * 2 + lax.axis_index("c")
        base_in = wid * (_RPW * _NUM_KY)
        base_out = wid * (_RPW * _ZERO_FILL_WIDTH)
        in_v = (in_v0, in_v1)
        out_v = (out_v0, out_v1)
        sem_in = (si0, si1)
        sem_out = (so0, so1)

        pltpu.sync_copy(idx_hbm, idx_v)

        zeros = jnp.zeros((16,), jnp.float32)

        for b in range(2):
            @plsc.parallel_loop(0, _OUT_W, step=16, unroll=8)
            def _(o, b=b):
                out_v[b][pl.ds(o, 16)] = zeros

        def in_copy(c, b):
            return pltpu.make_async_copy(
                x_hbm.at[pl.ds(base_in + c * _IN_W, _IN_W)], in_v[b],
                sem_in[b])

        def out_copy(c, b):
            return pltpu.make_async_copy(
                out_v[b], out_hbm.at[pl.ds(base_out + c * _OUT_W, _OUT_W)],
                sem_out[b])

        in_copy(0, 0).start()
        for c in range(_NCHUNK):
            b = c % 2
            if c + 1 < _NCHUNK:
                in_copy(c + 1, 1 - b).start()
            in_copy(c, b).wait()
            if c >= 2:
                out_copy(c - 2, b).wait()

            @plsc.parallel_loop(0, _IN_W, step=16, unroll=8)
            def _(o, b=b):
                v = in_v[b][pl.ds(o, 16)]
                ix = idx_v[pl.ds(o, 16)]
                plsc.store_scatter(out_v[b], [ix], v)

            out_copy(c, b).start()
        out_copy(_NCHUNK - 2, (_NCHUNK - 2) % 2).wait()
        out_copy(_NCHUNK - 1, (_NCHUNK - 1) % 2).wait()

    return k(x_flat, idx_flat)


def kernel(undersampled_ksp):
    lead = undersampled_ksp.shape[:-1]
    x_flat = undersampled_ksp.reshape(-1)
    out = _sc_zero_fill(x_flat, _chunk_scatter_indices())
    return out.reshape(*lead, _ZERO_FILL_WIDTH)


# speedup vs baseline: 5.0731x; 3.4348x over previous
import functools

import jax
import jax.numpy as jnp
import numpy as np
from jax import lax
from jax.experimental import pallas as pl
from jax.experimental.pallas import tpu as pltpu
from jax.experimental.pallas import tpu_sc as plsc

_ACCEL_FACTOR = 4
_NUM_CENTRAL_LINES = 30
_ZERO_FILL_WIDTH = 368


def _ky_positions():
    center = _ZERO_FILL_WIDTH // 2
    half_width = _NUM_CENTRAL_LINES // 2
    central = np.arange(center - half_width,
                        center + half_width + _NUM_CENTRAL_LINES % 2)
    accel = np.arange(_ZERO_FILL_WIDTH)[::_ACCEL_FACTOR]
    accel = accel[~np.isin(accel, central)]
    return np.sort(np.concatenate([central, accel]))


_KY = _ky_positions()
_NUM_KY = _KY.shape[0]

_SLABS = 32 * 15
_SLAB_ROWS = 320
_NW = 32
_SPW = _SLABS // _NW
_C = 80
_CPS = _SLAB_ROWS // _C
_NCHUNK = _SPW * _CPS

_COL_STARTS = [0, 16, 32, 48, 64, 80, 96, _NUM_KY - 16]


@jax.jit
def _sc_zero_fill(x3d):
    mesh = plsc.VectorSubcoreMesh(core_axis_name="c", subcore_axis_name="s")

    @functools.partial(
        pl.kernel,
        mesh=mesh,
        out_type=jax.ShapeDtypeStruct((_SLABS, _SLAB_ROWS, _ZERO_FILL_WIDTH),
                                      jnp.float32),
        compiler_params=pltpu.CompilerParams(needs_layout_passes=False),
        scratch_types=[
            pltpu.VMEM((_C, _NUM_KY), jnp.float32),
            pltpu.VMEM((_C, _NUM_KY), jnp.float32),
            pltpu.VMEM((_C, _ZERO_FILL_WIDTH), jnp.float32),
            pltpu.VMEM((_C, _ZERO_FILL_WIDTH), jnp.float32),
            pltpu.SemaphoreType.DMA,
            pltpu.SemaphoreType.DMA,
            pltpu.SemaphoreType.DMA,
            pltpu.SemaphoreType.DMA,
        ],
    )
    def k(x_hbm, out_hbm, in_v0, in_v1, out_v0, out_v1, si0, si1, so0, so1):
        wid = lax.axis_index("s") * 2 + lax.axis_index("c")
        slab0 = wid * _SPW
        in_v = (in_v0, in_v1)
        out_v = (out_v0, out_v1)
        sem_in = (si0, si1)
        sem_out = (so0, so1)

        zeros = jnp.zeros((16,), jnp.float32)

        for b in range(2):
            @plsc.parallel_loop(0, _C, step=1, unroll=2)
            def _(r, b=b):
                for t in range(_ZERO_FILL_WIDTH // 16):
                    out_v[b][r, pl.ds(t * 16, 16)] = zeros

        iota = lax.iota(jnp.int32, 16)

        def ky_of(j):
            return jnp.where(
                j < 43, 4 * j, jnp.where(j < 73, j + 126, 4 * j - 92))

        col_ix = [ky_of(s + iota) for s in _COL_STARTS]

        def in_copy(g, b):
            slab, r0 = slab0 + g // _CPS, (g % _CPS) * _C
            return pltpu.make_async_copy(
                x_hbm.at[slab, pl.ds(r0, _C), :], in_v[b], sem_in[b])

        def out_copy(g, b):
            slab, r0 = slab0 + g // _CPS, (g % _CPS) * _C
            return pltpu.make_async_copy(
                out_v[b], out_hbm.at[slab, pl.ds(r0, _C), :], sem_out[b])

        in_copy(0, 0).start()
        for g in range(_NCHUNK):
            b = g % 2
            if g + 1 < _NCHUNK:
                in_copy(g + 1, 1 - b).start()
            in_copy(g, b).wait()
            if g >= 2:
                out_copy(g - 2, b).wait()

            @plsc.parallel_loop(0, _C, step=1, unroll=2)
            def _(r, b=b):
                rr = jnp.full((16,), r, dtype=jnp.int32)
                for t, s in enumerate(_COL_STARTS):
                    v = in_v[b][r, pl.ds(s, 16)]
                    plsc.store_scatter(out_v[b], [rr, col_ix[t]], v)

            out_copy(g, b).start()
        out_copy(_NCHUNK - 2, (_NCHUNK - 2) % 2).wait()
        out_copy(_NCHUNK - 1, (_NCHUNK - 1) % 2).wait()

    return k(x3d)


def kernel(undersampled_ksp):
    lead = undersampled_ksp.shape[:-1]
    x3d = undersampled_ksp.reshape(_SLABS, _SLAB_ROWS, _NUM_KY)
    out = _sc_zero_fill(x3d)
    return out.reshape(*lead, _ZERO_FILL_WIDTH)


# speedup vs baseline: 5.4669x; 1.0776x over previous
import functools

import jax
import jax.numpy as jnp
import numpy as np
from jax import lax
from jax.experimental import pallas as pl
from jax.experimental.pallas import tpu as pltpu
from jax.experimental.pallas import tpu_sc as plsc

_ACCEL_FACTOR = 4
_NUM_CENTRAL_LINES = 30
_ZERO_FILL_WIDTH = 368


def _ky_positions():
    center = _ZERO_FILL_WIDTH // 2
    half_width = _NUM_CENTRAL_LINES // 2
    central = np.arange(center - half_width,
                        center + half_width + _NUM_CENTRAL_LINES % 2)
    accel = np.arange(_ZERO_FILL_WIDTH)[::_ACCEL_FACTOR]
    accel = accel[~np.isin(accel, central)]
    return np.sort(np.concatenate([central, accel]))


_KY = _ky_positions()
_NUM_KY = _KY.shape[0]

_SLABS = 32 * 15
_SLAB_ROWS = 320
_NW = 32
_SPW = _SLABS // _NW
_C = 80
_CPS = _SLAB_ROWS // _C
_NCHUNK = _SPW * _CPS

_COL_STARTS = [0, 16, 32, 48, 64, 80, 96, _NUM_KY - 16]


@jax.jit
def _sc_zero_fill(x3d):
    mesh = plsc.VectorSubcoreMesh(core_axis_name="c", subcore_axis_name="s")

    @functools.partial(
        pl.kernel,
        mesh=mesh,
        out_type=jax.ShapeDtypeStruct((_SLABS, _SLAB_ROWS, _ZERO_FILL_WIDTH),
                                      jnp.float32),
        compiler_params=pltpu.CompilerParams(needs_layout_passes=False),
        scratch_types=[
            pltpu.VMEM((_C, _NUM_KY), jnp.float32),
            pltpu.VMEM((_C, _NUM_KY), jnp.float32),
            pltpu.VMEM((_C, _ZERO_FILL_WIDTH), jnp.float32),
            pltpu.VMEM((_C, _ZERO_FILL_WIDTH), jnp.float32),
            pltpu.SemaphoreType.DMA,
            pltpu.SemaphoreType.DMA,
            pltpu.SemaphoreType.DMA,
            pltpu.SemaphoreType.DMA,
        ],
    )
    def k(x_hbm, out_hbm, in_v0, in_v1, out_v0, out_v1, si0, si1, so0, so1):
        wid = lax.axis_index("s") * 2 + lax.axis_index("c")
        slab0 = wid * _SPW
        in_v = (in_v0, in_v1)
        out_v = (out_v0, out_v1)
        sem_in = (si0, si1)
        sem_out = (so0, so1)

        zeros = jnp.zeros((16,), jnp.float32)

        for b in range(2):
            @plsc.parallel_loop(0, _C, step=1, unroll=2)
            def _(r, b=b):
                for t in range(_ZERO_FILL_WIDTH // 16):
                    out_v[b][r, pl.ds(t * 16, 16)] = zeros

        iota = lax.iota(jnp.int32, 16)

        def ky_of(j):
            return jnp.where(
                j < 43, 4 * j, jnp.where(j < 73, j + 126, 4 * j - 92))

        col_ix = [ky_of(s + iota) for s in _COL_STARTS]

        def in_copy(g, b):
            slab, r0 = slab0 + g // _CPS, (g % _CPS) * _C
            return pltpu.make_async_copy(
                x_hbm.at[slab, pl.ds(r0, _C), :], in_v[b], sem_in[b])

        def out_copy(g, b):
            slab, r0 = slab0 + g // _CPS, (g % _CPS) * _C
            return pltpu.make_async_copy(
                out_v[b], out_hbm.at[slab, pl.ds(r0, _C), :], sem_out[b])

        def scatter_chunk(b):
            @plsc.parallel_loop(0, _C, step=1, unroll=2)
            def _(r):
                rr = jnp.full((16,), r, dtype=jnp.int32)
                for t, s in enumerate(_COL_STARTS):
                    v = in_v[b][r, pl.ds(s, 16)]
                    plsc.store_scatter(out_v[b], [rr, col_ix[t]], v)

        in_copy(0, 0).start()
        for g in range(2):
            b = g % 2
            in_copy(g + 1, 1 - b).start()
            in_copy(g, b).wait()
            scatter_chunk(b)
            out_copy(g, b).start()

        def pair(i, carry):
            for b in range(2):
                g = 2 * i + b
                in_copy(g + 1, 1 - b).start()
                in_copy(g, b).wait()
                out_copy(g - 2, b).wait()
                scatter_chunk(b)
                out_copy(g, b).start()
            return carry

        lax.fori_loop(1, _NCHUNK // 2 - 1, pair, 0)

        for g in range(_NCHUNK - 2, _NCHUNK):
            b = g % 2
            if g + 1 < _NCHUNK:
                in_copy(g + 1, 1 - b).start()
            in_copy(g, b).wait()
            out_copy(g - 2, b).wait()
            scatter_chunk(b)
            out_copy(g, b).start()
        out_copy(_NCHUNK - 2, (_NCHUNK - 2) % 2).wait()
        out_copy(_NCHUNK - 1, (_NCHUNK - 1) % 2).wait()

    return k(x3d)


def kernel(undersampled_ksp):
    lead = undersampled_ksp.shape[:-1]
    x3d = undersampled_ksp.reshape(_SLABS, _SLAB_ROWS, _NUM_KY)
    out = _sc_zero_fill(x3d)
    return out.reshape(*lead, _ZERO_FILL_WIDTH)
